# baseline (speedup 1.0000x reference)
import functools
import jax
import jax.numpy as jnp
from jax import lax
from jax.experimental import pallas as pl
from jax.experimental.pallas import tpu as pltpu
from jax.experimental.pallas import tpu_sc as plsc

_NC, _NS = 2, 16
_C = 8
_NBUF = 3


def kernel(input_embeddings, table):
    B, S, D = input_embeddings.shape
    BS = B * S
    NW = _NC * _NS
    s_per_w = S // NW
    chunks = s_per_w // _C
    BC = B * _C
    main = (chunks // _NBUF) * _NBUF

    mesh = plsc.VectorSubcoreMesh(core_axis_name="c", subcore_axis_name="s")

    @functools.partial(
        pl.kernel,
        mesh=mesh,
        out_type=jax.ShapeDtypeStruct((BS, D), jnp.float32),
        scratch_types=(
            [pltpu.VMEM((BC, D), jnp.float32)] * _NBUF
            + [pltpu.VMEM((_C, D), jnp.float32)] * _NBUF
            + [pltpu.SemaphoreType.DMA] * (2 * _NBUF)
        ),
        compiler_params=pltpu.CompilerParams(use_tc_tiling_on_sc=True),
    )
    def sc_add(x_hbm, t_hbm, out_hbm, *bufs):
        xbufs = bufs[0:_NBUF]
        tbufs = bufs[_NBUF:2 * _NBUF]
        isems = bufs[2 * _NBUF:3 * _NBUF]
        osems = bufs[3 * _NBUF:4 * _NBUF]
        wid = lax.axis_index("s") * _NC + lax.axis_index("c")
        s0 = wid * s_per_w

        def start_in(j, st):
            sbase = s0 + j * _C
            pltpu.async_copy(t_hbm.at[pl.ds(sbase, _C)], tbufs[st], isems[st])
            for bb in range(B):
                pltpu.async_copy(
                    x_hbm.at[pl.ds(bb * S + sbase, _C)],
                    xbufs[st].at[pl.ds(bb * _C, _C)], isems[st])

        def wait_in(j, st):
            sbase = s0 + j * _C
            pltpu.make_async_copy(
                t_hbm.at[pl.ds(sbase, _C)], tbufs[st], isems[st]).wait()
            for bb in range(B):
                pltpu.make_async_copy(
                    x_hbm.at[pl.ds(bb * S + sbase, _C)],
                    xbufs[st].at[pl.ds(bb * _C, _C)], isems[st]).wait()

        def start_out(j, st):
            sbase = s0 + j * _C
            for bb in range(B):
                pltpu.async_copy(
                    xbufs[st].at[pl.ds(bb * _C, _C)],
                    out_hbm.at[pl.ds(bb * S + sbase, _C)], osems[st])

        def wait_out(j, st):
            sbase = s0 + j * _C
            for bb in range(B):
                pltpu.make_async_copy(
                    xbufs[st].at[pl.ds(bb * _C, _C)],
                    out_hbm.at[pl.ds(bb * S + sbase, _C)], osems[st]).wait()

        def step(j, st):
            @pl.when(j + 2 < chunks)
            def _():
                @pl.when(j >= 1)
                def _():
                    wait_out(j - 1, (st + 2) % _NBUF)
                start_in(j + 2, (st + 2) % _NBUF)

            wait_in(j, st)
            xb, tb = xbufs[st], tbufs[st]

            @plsc.parallel_loop(0, D, 16, unroll=4)
            def _(i):
                sl = pl.ds(i, 16)
                for r in range(BC):
                    xb[r, sl] = xb[r, sl] + tb[r % _C, sl]

            start_out(j, st)

        start_in(0, 0)
        start_in(1, 1)

        def body(jj, carry):
            for off in range(_NBUF):
                step(jj * _NBUF + off, off)
            return carry

        lax.fori_loop(0, main // _NBUF, body, 0)
        for j in range(main, chunks):
            step(j, j % _NBUF)
        for j in range(chunks - _NBUF, chunks):
            wait_out(j, j % _NBUF)

    out = sc_add(input_embeddings.reshape(BS, D), table)
    return out.reshape(B, S, D)


# speedup vs baseline: 1.0431x; 1.0431x over previous
import functools
import jax
import jax.numpy as jnp
from jax import lax
from jax.experimental import pallas as pl
from jax.experimental.pallas import tpu as pltpu
from jax.experimental.pallas import tpu_sc as plsc

_NC, _NS = 2, 16
_C = 8


def kernel(input_embeddings, table):
    B, S, D = input_embeddings.shape
    BS = B * S
    NW = _NC * _NS
    s_per_w = S // NW
    chunks = s_per_w // _C
    BC = B * _C

    mesh = plsc.VectorSubcoreMesh(core_axis_name="c", subcore_axis_name="s")

    @functools.partial(
        pl.kernel,
        mesh=mesh,
        out_type=jax.ShapeDtypeStruct((BS, D), jnp.float32),
        scratch_types=[
            pltpu.VMEM((BC, D), jnp.float32),
            pltpu.VMEM((BC, D), jnp.float32),
            pltpu.VMEM((_C, D), jnp.float32),
            pltpu.VMEM((_C, D), jnp.float32),
            pltpu.SemaphoreType.DMA,
            pltpu.SemaphoreType.DMA,
            pltpu.SemaphoreType.DMA,
            pltpu.SemaphoreType.DMA,
        ],
        compiler_params=pltpu.CompilerParams(use_tc_tiling_on_sc=True),
    )
    def sc_add(x_hbm, t_hbm, out_hbm, xb0, xb1, tb0, tb1, is0, is1, os0, os1):
        wid = lax.axis_index("s") * _NC + lax.axis_index("c")
        s0 = wid * s_per_w
        xbufs, tbufs = (xb0, xb1), (tb0, tb1)
        isems, osems = (is0, is1), (os0, os1)

        def start_in(j, b):
            sbase = s0 + j * _C
            pltpu.async_copy(t_hbm.at[pl.ds(sbase, _C)], tbufs[b], isems[b])
            for bb in range(B):
                pltpu.async_copy(
                    x_hbm.at[pl.ds(bb * S + sbase, _C)],
                    xbufs[b].at[pl.ds(bb * _C, _C)], isems[b])

        def wait_in(j, b):
            sbase = s0 + j * _C
            pltpu.make_async_copy(
                t_hbm.at[pl.ds(sbase, _C)], tbufs[b], isems[b]).wait()
            for bb in range(B):
                pltpu.make_async_copy(
                    x_hbm.at[pl.ds(bb * S + sbase, _C)],
                    xbufs[b].at[pl.ds(bb * _C, _C)], isems[b]).wait()

        def start_out(j, b):
            sbase = s0 + j * _C
            for bb in range(B):
                pltpu.async_copy(
                    xbufs[b].at[pl.ds(bb * _C, _C)],
                    out_hbm.at[pl.ds(bb * S + sbase, _C)], osems[b])

        def wait_out(j, b):
            sbase = s0 + j * _C
            for bb in range(B):
                pltpu.make_async_copy(
                    xbufs[b].at[pl.ds(bb * _C, _C)],
                    out_hbm.at[pl.ds(bb * S + sbase, _C)], osems[b]).wait()

        start_in(0, 0)

        def half_step(jj, b):
            j = jj * 2 + b
            xb, tb = xbufs[b], tbufs[b]

            @pl.when(j + 1 < chunks)
            def _():
                @pl.when(j >= 1)
                def _():
                    wait_out(j - 1, 1 - b)
                start_in(j + 1, 1 - b)

            wait_in(j, b)

            @plsc.parallel_loop(0, D, 16, unroll=4)
            def _(i):
                sl = pl.ds(i, 16)
                for r in range(BC):
                    xb[r, sl] = xb[r, sl] + tb[r % _C, sl]

            start_out(j, b)

        def body(jj, carry):
            half_step(jj, 0)
            half_step(jj, 1)
            return carry

        lax.fori_loop(0, chunks // 2, body, 0)
        wait_out(chunks - 2, 0)
        wait_out(chunks - 1, 1)

    out = sc_add(input_embeddings.reshape(BS, D), table)
    return out.reshape(B, S, D)
